# baseline (speedup 1.0000x reference)
import functools

import jax
import jax.numpy as jnp
import numpy as np
from jax.experimental import pallas as pl
from jax.experimental.pallas import tpu as pltpu
from jax.sharding import Mesh, PartitionSpec as P

_MARGIN = 1.0
_BR = 256


def _hard_triplet_kernel(off_ref, et_ref, labc_ref, labr_ref,
                         loss_ref, pos_ref, neg_ref, sq_ref):
    i = pl.program_id(0)
    n = et_ref.shape[1]
    off = off_ref[0, 0]

    @pl.when(i == 0)
    def _():
        et = et_ref[...]
        sq_ref[...] = jnp.sum(et * et, axis=0, keepdims=True)

    r0 = pl.multiple_of(off + i * _BR, _BR)
    lhs = et_ref[:, pl.ds(r0, _BR)] * (-2.0)
    g2 = jax.lax.dot_general(lhs, et_ref[...], (((0,), (0,)), ((), ())),
                             preferred_element_type=jnp.float32)
    c = sq_ref[...] + g2

    lab_i = labc_ref[pl.ds(r0, _BR), :]
    same = lab_i == labr_ref[...]

    inf = jnp.inf
    pos_c = jnp.where(same, c, -inf)
    neg_c = jnp.where(same, inf, c)

    pmax = jnp.max(pos_c, axis=1, keepdims=True)
    nmin = jnp.min(neg_c, axis=1, keepdims=True)

    iota_f = jax.lax.broadcasted_iota(jnp.int32, (1, n), 1).astype(jnp.float32)
    big = jnp.float32(n)
    pidx_f = jnp.min(jnp.where(pos_c == pmax, iota_f, big), axis=1,
                     keepdims=True)
    nidx_f = jnp.min(jnp.where(neg_c == nmin, iota_f, big), axis=1,
                     keepdims=True)
    pos_ref[...] = pidx_f.astype(jnp.int32)
    neg_ref[...] = nidx_f.astype(jnp.int32)

    losses = jax.nn.relu(pmax - nmin + _MARGIN)
    s = jnp.sum(losses)
    loss_ref[0, 0] = jnp.where(i == 0, s, loss_ref[0, 0] + s)


def _shard_body(rows_per, et, labc, labr):
    n = et.shape[1]
    d = et.shape[0]
    off = (jax.lax.axis_index("x") * rows_per).astype(jnp.int32)
    off = off.reshape(1, 1)
    grid = (rows_per // _BR,)
    return pl.pallas_call(
        _hard_triplet_kernel,
        grid=grid,
        in_specs=[
            pl.BlockSpec(memory_space=pltpu.SMEM),
            pl.BlockSpec((d, n), lambda i: (0, 0)),
            pl.BlockSpec((n, 1), lambda i: (0, 0)),
            pl.BlockSpec((1, n), lambda i: (0, 0)),
        ],
        out_specs=[
            pl.BlockSpec(memory_space=pltpu.SMEM),
            pl.BlockSpec((_BR, 1), lambda i: (i, 0)),
            pl.BlockSpec((_BR, 1), lambda i: (i, 0)),
        ],
        out_shape=[
            jax.ShapeDtypeStruct((1, 1), jnp.float32),
            jax.ShapeDtypeStruct((rows_per, 1), jnp.int32),
            jax.ShapeDtypeStruct((rows_per, 1), jnp.int32),
        ],
        scratch_shapes=[pltpu.VMEM((1, n), jnp.float32)],
        compiler_params=pltpu.CompilerParams(
            dimension_semantics=("arbitrary",),
        ),
    )(off, et, labc, labr)


def kernel(query_embeddings, query_target, db_embeddings, db_target):
    n, d = db_embeddings.shape
    labc = db_target.astype(jnp.int32).reshape(n, 1)
    labr = db_target.astype(jnp.int32).reshape(1, n)
    et = db_embeddings.T

    devs = jax.devices()
    ndev = 2 if len(devs) >= 2 and n % (2 * _BR) == 0 else 1
    rows_per = n // ndev
    mesh = Mesh(np.array(devs[:ndev]), ("x",))
    body = jax.shard_map(
        functools.partial(_shard_body, rows_per),
        mesh=mesh,
        in_specs=(P(None, None), P(None, None), P(None, None)),
        out_specs=(P("x", None), P("x", None), P("x", None)),
        check_vma=False,
    )
    loss_parts, pos, neg = body(et, labc, labr)

    loss = jnp.sum(loss_parts) / n
    anchors = jnp.arange(n, dtype=jnp.int32)
    triplets = jnp.stack([anchors, pos[:, 0], neg[:, 0]], axis=1)
    return (loss, triplets)


# speedup vs baseline: 1.7231x; 1.7231x over previous
import functools

import jax
import jax.numpy as jnp
import numpy as np
from jax.experimental import pallas as pl
from jax.experimental.pallas import tpu as pltpu
from jax.sharding import Mesh, PartitionSpec as P

_MARGIN = 1.0
_BR = 256


def _hard_triplet_kernel(off_ref, db_ref, labc_ref, labr_ref,
                         loss_ref, pos_ref, neg_ref, et_ref, sq_ref):
    i = pl.program_id(0)
    n = db_ref.shape[0]
    off = off_ref[0, 0]

    @pl.when(i == 0)
    def _():
        et = db_ref[...].T
        et_ref[...] = et
        sq_ref[...] = jnp.sum(et * et, axis=0, keepdims=True)

    r0 = pl.multiple_of(off + i * _BR, _BR)
    lhs = db_ref[pl.ds(r0, _BR), :] * (-2.0)
    g2 = jax.lax.dot_general(lhs, et_ref[...], (((1,), (0,)), ((), ())),
                             preferred_element_type=jnp.float32)
    c = sq_ref[...] + g2

    lab_i = labc_ref[pl.ds(r0, _BR), :]
    same = lab_i == labr_ref[...]

    inf = jnp.inf
    pos_c = jnp.where(same, c, -inf)
    neg_c = jnp.where(same, inf, c)

    pmax = jnp.max(pos_c, axis=1, keepdims=True)
    nmin = jnp.min(neg_c, axis=1, keepdims=True)

    iota_f = jax.lax.broadcasted_iota(jnp.int32, (1, n), 1).astype(jnp.float32)
    big = jnp.float32(n)
    pidx_f = jnp.min(jnp.where(pos_c == pmax, iota_f, big), axis=1,
                     keepdims=True)
    nidx_f = jnp.min(jnp.where(neg_c == nmin, iota_f, big), axis=1,
                     keepdims=True)
    pos_ref[...] = pidx_f.astype(jnp.int32)
    neg_ref[...] = nidx_f.astype(jnp.int32)

    losses = jax.nn.relu(pmax - nmin + _MARGIN)
    s = jnp.sum(losses)
    loss_ref[0, 0] = jnp.where(i == 0, s, loss_ref[0, 0] + s)


def _shard_body(axis, rows_per, db, labc, labr):
    n, d = db.shape
    off = (jax.lax.axis_index(axis) * rows_per).astype(jnp.int32)
    off = off.reshape(1, 1)
    grid = (rows_per // _BR,)
    return pl.pallas_call(
        _hard_triplet_kernel,
        grid=grid,
        in_specs=[
            pl.BlockSpec(memory_space=pltpu.SMEM),
            pl.BlockSpec((n, d), lambda i: (0, 0)),
            pl.BlockSpec((n, 1), lambda i: (0, 0)),
            pl.BlockSpec((1, n), lambda i: (0, 0)),
        ],
        out_specs=[
            pl.BlockSpec(memory_space=pltpu.SMEM),
            pl.BlockSpec((_BR, 1), lambda i: (i, 0)),
            pl.BlockSpec((_BR, 1), lambda i: (i, 0)),
        ],
        out_shape=[
            jax.ShapeDtypeStruct((1, 1), jnp.float32),
            jax.ShapeDtypeStruct((rows_per, 1), jnp.int32),
            jax.ShapeDtypeStruct((rows_per, 1), jnp.int32),
        ],
        scratch_shapes=[pltpu.VMEM((d, n), jnp.float32),
                        pltpu.VMEM((1, n), jnp.float32)],
        compiler_params=pltpu.CompilerParams(
            dimension_semantics=("arbitrary",),
        ),
    )(off, db, labc, labr)


def kernel(query_embeddings, query_target, db_embeddings, db_target):
    n, d = db_embeddings.shape
    labc = db_target.astype(jnp.int32).reshape(n, 1)
    labr = db_target.astype(jnp.int32).reshape(1, n)

    ctx_mesh = jax.sharding.get_abstract_mesh()
    if (ctx_mesh is not None and not ctx_mesh.empty
            and n % (ctx_mesh.size * _BR) == 0):
        mesh = ctx_mesh
        axis = mesh.axis_names[0]
        ndev = mesh.size
    else:
        devs = jax.devices()
        ndev = 2 if len(devs) >= 2 and n % (2 * _BR) == 0 else 1
        axis = "x"
        mesh = jax.make_mesh((ndev,), (axis,),
                             axis_types=(jax.sharding.AxisType.Auto,),
                             devices=devs[:ndev])
    rows_per = n // ndev
    body = jax.shard_map(
        functools.partial(_shard_body, axis, rows_per),
        mesh=mesh,
        in_specs=(P(None, None), P(None, None), P(None, None)),
        out_specs=(P(axis, None), P(axis, None), P(axis, None)),
        check_vma=False,
    )
    loss_parts, pos, neg = body(db_embeddings, labc, labr)

    loss = jnp.sum(loss_parts) / n
    anchors = jnp.arange(n, dtype=jnp.int32)
    triplets = jnp.stack([anchors, pos[:, 0], neg[:, 0]], axis=1)
    return (loss, triplets)
